# baseline (device time: 179039 ns/iter reference)
import math

import jax
import jax.numpy as jnp
from jax import lax
from jax.experimental import pallas as pl
from jax.experimental.pallas import tpu as pltpu

N_DEV = 4
TQ = 512


def kernel(q, k, v):
    s_per, d = q.shape
    scale = 1.0 / math.sqrt(float(d))
    n_q_tiles = s_per // TQ
    n_hops = N_DEV - 1

    def body(q_ref, k_ref, v_ref, out_ref, qb_ref, k0_ref, v0_ref,
             k_comm, v_comm, l_ref, ksend, krecv, vsend, vrecv):
        my = lax.axis_index("i")
        left = lax.rem(my + N_DEV - 1, N_DEV)
        right = lax.rem(my + 1, N_DEV)

        qb_ref[:, :] = (q_ref[:, :] * (scale * 1.4426950408889634)).astype(
            jnp.bfloat16)
        k0_ref[:, :] = k_ref[:, :].astype(jnp.bfloat16)
        v0_ref[:, :] = v_ref[:, :].astype(jnp.bfloat16)

        barrier = pltpu.get_barrier_semaphore()
        for nbr in (left, right):
            pl.semaphore_signal(barrier, inc=1, device_id=(nbr,),
                                device_id_type=pl.DeviceIdType.MESH)
        pl.semaphore_wait(barrier, 2)

        def make_hop(h):
            src_k = k0_ref if h == 0 else k_comm.at[h - 1]
            src_v = v0_ref if h == 0 else v_comm.at[h - 1]
            rk = pltpu.make_async_remote_copy(
                src_ref=src_k, dst_ref=k_comm.at[h],
                send_sem=ksend.at[h], recv_sem=krecv.at[h],
                device_id=(right,), device_id_type=pl.DeviceIdType.MESH)
            rv = pltpu.make_async_remote_copy(
                src_ref=src_v, dst_ref=v_comm.at[h],
                send_sem=vsend.at[h], recv_sem=vrecv.at[h],
                device_id=(right,), device_id_type=pl.DeviceIdType.MESH)
            return rk, rv

        def process_block(kb, vb, is_first, is_last):
            def tile_body(t, carry):
                rows = pl.ds(t * TQ, TQ)
                qt = qb_ref[rows, :]
                s = lax.dot_general(
                    qt, kb, (((1,), (1,)), ((), ())),
                    preferred_element_type=jnp.float32)
                p = jnp.exp2(s)
                l_add = jnp.sum(p, axis=1, keepdims=True)
                pv = lax.dot_general(
                    p.astype(jnp.bfloat16), vb, (((1,), (0,)), ((), ())),
                    preferred_element_type=jnp.float32)
                if is_first:
                    l_new = l_add
                    acc = pv
                else:
                    l_new = l_ref[rows, :] + l_add
                    acc = out_ref[rows, :] + pv
                if is_last:
                    out_ref[rows, :] = acc / l_new
                else:
                    out_ref[rows, :] = acc
                    l_ref[rows, :] = l_new
                return carry

            lax.fori_loop(0, n_q_tiles, tile_body, 0)

        hops = [make_hop(0)]
        hops[0][0].start()
        hops[0][1].start()
        process_block(k0_ref[:, :], v0_ref[:, :], is_first=True, is_last=False)

        for h in range(n_hops):
            rk, rv = hops[h]
            rk.wait_recv()
            rv.wait_recv()
            if h + 1 < n_hops:
                nxt = make_hop(h + 1)
                nxt[0].start()
                nxt[1].start()
                hops.append(nxt)
            process_block(k_comm[h, :, :], v_comm[h, :, :],
                          is_first=False, is_last=(h == n_hops - 1))

        for rk, rv in hops:
            rk.wait_send()
            rv.wait_send()

    return pl.pallas_call(
        body,
        out_shape=jax.ShapeDtypeStruct((s_per, d), jnp.float32),
        in_specs=[pl.BlockSpec(memory_space=pltpu.VMEM)] * 3,
        out_specs=pl.BlockSpec(memory_space=pltpu.VMEM),
        scratch_shapes=[
            pltpu.VMEM((s_per, d), jnp.bfloat16),
            pltpu.VMEM((s_per, d), jnp.bfloat16),
            pltpu.VMEM((s_per, d), jnp.bfloat16),
            pltpu.VMEM((n_hops, s_per, d), jnp.bfloat16),
            pltpu.VMEM((n_hops, s_per, d), jnp.bfloat16),
            pltpu.VMEM((s_per, 1), jnp.float32),
            pltpu.SemaphoreType.DMA((n_hops,)),
            pltpu.SemaphoreType.DMA((n_hops,)),
            pltpu.SemaphoreType.DMA((n_hops,)),
            pltpu.SemaphoreType.DMA((n_hops,)),
        ],
        compiler_params=pltpu.CompilerParams(
            collective_id=0,
            vmem_limit_bytes=100 * 1024 * 1024,
        ),
    )(q, k, v)


# device time: 166566 ns/iter; 1.0749x vs baseline; 1.0749x over previous
import math

import jax
import jax.numpy as jnp
from jax import lax
from jax.experimental import pallas as pl
from jax.experimental.pallas import tpu as pltpu

N_DEV = 4
TQ = 512
N_CHUNK = 2


def kernel(q, k, v):
    s_per, d = q.shape
    scale = 1.0 / math.sqrt(float(d))
    n_q_tiles = s_per // TQ
    n_hops = N_DEV - 1
    rc = s_per // N_CHUNK

    def body(q_ref, k_ref, v_ref, out_ref, qb_ref, k0_ref, v0_ref,
             k_comm, v_comm, l_ref, ksend, krecv, vsend, vrecv):
        my = lax.axis_index("i")
        left = lax.rem(my + N_DEV - 1, N_DEV)
        right = lax.rem(my + 1, N_DEV)

        qb_ref[:, :] = (q_ref[:, :] * (scale * 1.4426950408889634)).astype(
            jnp.bfloat16)
        k0_ref[:, :] = k_ref[:, :].astype(jnp.bfloat16)
        v0_ref[:, :] = v_ref[:, :].astype(jnp.bfloat16)

        barrier = pltpu.get_barrier_semaphore()
        for nbr in (left, right):
            pl.semaphore_signal(barrier, inc=1, device_id=(nbr,),
                                device_id_type=pl.DeviceIdType.MESH)
        pl.semaphore_wait(barrier, 2)

        def make_hop(h, c):
            rows = pl.ds(c * rc, rc)
            src_k = k0_ref.at[rows] if h == 0 else k_comm.at[h - 1, rows]
            src_v = v0_ref.at[rows] if h == 0 else v_comm.at[h - 1, rows]
            rk = pltpu.make_async_remote_copy(
                src_ref=src_k, dst_ref=k_comm.at[h, rows],
                send_sem=ksend.at[h, c], recv_sem=krecv.at[h, c],
                device_id=(right,), device_id_type=pl.DeviceIdType.MESH)
            rv = pltpu.make_async_remote_copy(
                src_ref=src_v, dst_ref=v_comm.at[h, rows],
                send_sem=vsend.at[h, c], recv_sem=vrecv.at[h, c],
                device_id=(right,), device_id_type=pl.DeviceIdType.MESH)
            return rk, rv

        def process_chunk(kb, vb, is_first, is_last):
            def tile_body(t, carry):
                rows = pl.ds(t * TQ, TQ)
                qt = qb_ref[rows, :]
                s = lax.dot_general(
                    qt, kb, (((1,), (1,)), ((), ())),
                    preferred_element_type=jnp.float32)
                p = jnp.exp2(s)
                l_add = jnp.sum(p, axis=1, keepdims=True)
                pv = lax.dot_general(
                    p.astype(jnp.bfloat16), vb, (((1,), (0,)), ((), ())),
                    preferred_element_type=jnp.float32)
                if is_first:
                    l_new = l_add
                    acc = pv
                else:
                    l_new = l_ref[rows, :] + l_add
                    acc = out_ref[rows, :] + pv
                if is_last:
                    out_ref[rows, :] = acc / l_new
                else:
                    out_ref[rows, :] = acc
                    l_ref[rows, :] = l_new
                return carry

            lax.fori_loop(0, n_q_tiles, tile_body, 0)

        hops = {(0, c): make_hop(0, c) for c in range(N_CHUNK)}
        for c in range(N_CHUNK):
            hops[(0, c)][0].start()
            hops[(0, c)][1].start()
        process_chunk(k0_ref[:, :], v0_ref[:, :], is_first=True, is_last=False)

        for h in range(n_hops):
            for c in range(N_CHUNK):
                rk, rv = hops[(h, c)]
                rk.wait_recv()
                rv.wait_recv()
                if h + 1 < n_hops:
                    nxt = make_hop(h + 1, c)
                    nxt[0].start()
                    nxt[1].start()
                    hops[(h + 1, c)] = nxt
                process_chunk(
                    k_comm[h, c * rc:(c + 1) * rc, :],
                    v_comm[h, c * rc:(c + 1) * rc, :],
                    is_first=False,
                    is_last=(h == n_hops - 1 and c == N_CHUNK - 1))

        for rk, rv in hops.values():
            rk.wait_send()
            rv.wait_send()

    return pl.pallas_call(
        body,
        out_shape=jax.ShapeDtypeStruct((s_per, d), jnp.float32),
        in_specs=[pl.BlockSpec(memory_space=pltpu.VMEM)] * 3,
        out_specs=pl.BlockSpec(memory_space=pltpu.VMEM),
        scratch_shapes=[
            pltpu.VMEM((s_per, d), jnp.bfloat16),
            pltpu.VMEM((s_per, d), jnp.bfloat16),
            pltpu.VMEM((s_per, d), jnp.bfloat16),
            pltpu.VMEM((n_hops, s_per, d), jnp.bfloat16),
            pltpu.VMEM((n_hops, s_per, d), jnp.bfloat16),
            pltpu.VMEM((s_per, 1), jnp.float32),
            pltpu.SemaphoreType.DMA((n_hops, N_CHUNK)),
            pltpu.SemaphoreType.DMA((n_hops, N_CHUNK)),
            pltpu.SemaphoreType.DMA((n_hops, N_CHUNK)),
            pltpu.SemaphoreType.DMA((n_hops, N_CHUNK)),
        ],
        compiler_params=pltpu.CompilerParams(
            collective_id=0,
            vmem_limit_bytes=100 * 1024 * 1024,
        ),
    )(q, k, v)


# device time: 157954 ns/iter; 1.1335x vs baseline; 1.0545x over previous
import math

import jax
import jax.numpy as jnp
from jax import lax
from jax.experimental import pallas as pl
from jax.experimental.pallas import tpu as pltpu

N_DEV = 4
TQ = 512
N_CHUNK = 4


def kernel(q, k, v):
    s_per, d = q.shape
    scale = 1.0 / math.sqrt(float(d))
    n_q_tiles = s_per // TQ
    n_hops = N_DEV - 1
    rc = s_per // N_CHUNK

    def body(q_ref, k_ref, v_ref, out_ref, qb_ref, k0_ref, v0_ref,
             k_comm, v_comm, l_ref, ksend, krecv, vsend, vrecv):
        my = lax.axis_index("i")
        left = lax.rem(my + N_DEV - 1, N_DEV)
        right = lax.rem(my + 1, N_DEV)

        qb_ref[:, :] = (q_ref[:, :] * (scale * 1.4426950408889634)).astype(
            jnp.bfloat16)
        k0_ref[:, :] = k_ref[:, :].astype(jnp.bfloat16)
        v0_ref[:, :] = v_ref[:, :].astype(jnp.bfloat16)

        barrier = pltpu.get_barrier_semaphore()
        for nbr in (left, right):
            pl.semaphore_signal(barrier, inc=1, device_id=(nbr,),
                                device_id_type=pl.DeviceIdType.MESH)
        pl.semaphore_wait(barrier, 2)

        def make_hop(h, c):
            rows = pl.ds(c * rc, rc)
            src_k = k0_ref.at[rows] if h == 0 else k_comm.at[h - 1, rows]
            src_v = v0_ref.at[rows] if h == 0 else v_comm.at[h - 1, rows]
            rk = pltpu.make_async_remote_copy(
                src_ref=src_k, dst_ref=k_comm.at[h, rows],
                send_sem=ksend.at[h, c], recv_sem=krecv.at[h, c],
                device_id=(right,), device_id_type=pl.DeviceIdType.MESH)
            rv = pltpu.make_async_remote_copy(
                src_ref=src_v, dst_ref=v_comm.at[h, rows],
                send_sem=vsend.at[h, c], recv_sem=vrecv.at[h, c],
                device_id=(right,), device_id_type=pl.DeviceIdType.MESH)
            return rk, rv

        def process_chunk(kb, vb, is_first, is_last):
            def one_tile(rows):
                qt = qb_ref[rows, :]
                s = lax.dot_general(
                    qt, kb, (((1,), (1,)), ((), ())),
                    preferred_element_type=jnp.float32)
                p = jnp.exp2(s)
                l_add = jnp.sum(p, axis=1, keepdims=True)
                pv = lax.dot_general(
                    p.astype(jnp.bfloat16), vb, (((1,), (0,)), ((), ())),
                    preferred_element_type=jnp.float32)
                if is_first:
                    l_new = l_add
                    acc = pv
                else:
                    l_new = l_ref[rows, :] + l_add
                    acc = out_ref[rows, :] + pv
                if is_last:
                    out_ref[rows, :] = acc / l_new
                else:
                    out_ref[rows, :] = acc
                    l_ref[rows, :] = l_new

            def tile_body(t, carry):
                for u in range(4):
                    one_tile(pl.ds((t * 4 + u) * TQ, TQ))
                return carry

            lax.fori_loop(0, n_q_tiles // 4, tile_body, 0)

        hops = {(0, c): make_hop(0, c) for c in range(N_CHUNK)}
        for c in range(N_CHUNK):
            hops[(0, c)][0].start()
            hops[(0, c)][1].start()
        process_chunk(k0_ref[:, :], v0_ref[:, :], is_first=True, is_last=False)

        for h in range(n_hops):
            for c in range(N_CHUNK):
                rk, rv = hops[(h, c)]
                rk.wait_recv()
                rv.wait_recv()
                if h + 1 < n_hops:
                    nxt = make_hop(h + 1, c)
                    nxt[0].start()
                    nxt[1].start()
                    hops[(h + 1, c)] = nxt
                process_chunk(
                    k_comm[h, c * rc:(c + 1) * rc, :],
                    v_comm[h, c * rc:(c + 1) * rc, :],
                    is_first=False,
                    is_last=(h == n_hops - 1 and c == N_CHUNK - 1))

        for rk, rv in hops.values():
            rk.wait_send()
            rv.wait_send()

    return pl.pallas_call(
        body,
        out_shape=jax.ShapeDtypeStruct((s_per, d), jnp.float32),
        in_specs=[pl.BlockSpec(memory_space=pltpu.VMEM)] * 3,
        out_specs=pl.BlockSpec(memory_space=pltpu.VMEM),
        scratch_shapes=[
            pltpu.VMEM((s_per, d), jnp.bfloat16),
            pltpu.VMEM((s_per, d), jnp.bfloat16),
            pltpu.VMEM((s_per, d), jnp.bfloat16),
            pltpu.VMEM((n_hops, s_per, d), jnp.bfloat16),
            pltpu.VMEM((n_hops, s_per, d), jnp.bfloat16),
            pltpu.VMEM((s_per, 1), jnp.float32),
            pltpu.SemaphoreType.DMA((n_hops, N_CHUNK)),
            pltpu.SemaphoreType.DMA((n_hops, N_CHUNK)),
            pltpu.SemaphoreType.DMA((n_hops, N_CHUNK)),
            pltpu.SemaphoreType.DMA((n_hops, N_CHUNK)),
        ],
        compiler_params=pltpu.CompilerParams(
            collective_id=0,
            vmem_limit_bytes=100 * 1024 * 1024,
        ),
    )(q, k, v)
